# baseline (speedup 1.0000x reference)
import jax
import jax.numpy as jnp
from jax import lax
from jax.experimental import pallas as pl
from jax.experimental.pallas import tpu as pltpu
from jax.experimental.pallas import tpu_sc as plsc

B = 1024
S = 200
H = 128
SPLIT_A = 104
SPLIT_B = S - SPLIT_A
NC = 2
NS = 16
NW = NC * NS
ROWS_PER_W = B // NW
LANES = 16
KCH = H // LANES
NQ = S // LANES
TAIL = S - NQ * LANES
TEN_PAD = 104


def _body(ids_h, cidx_h, pos_h, typ_h, ten_h, word_h, out_h,
          pos_v, combo_v, typ_v, ten_v, acc_v, idx_a, idx_b, cidx_v, gsem):
    c = lax.axis_index("c")
    s = lax.axis_index("s")
    wid = s * NC + c

    pltpu.sync_copy(pos_h, pos_v)
    pltpu.sync_copy(typ_h, typ_v)
    pltpu.sync_copy(ten_h, ten_v)

    def build(t, carry):
        for half in range(2):
            for k in range(KCH):
                sl = pl.ds(k * LANES, LANES)
                combo_v[half * 100 + t, sl] = (
                    ten_v[t, sl] + typ_v[pl.ds(half * H + k * LANES, LANES)])
        return carry
    lax.fori_loop(0, 100, build, 0)

    def do_token(t, ct):
        for k in range(KCH):
            sl = pl.ds(k * LANES, LANES)
            acc_v[t, sl] = acc_v[t, sl] + pos_v[t, sl] + combo_v[ct, sl]

    def row(r, carry):
        b = wid * ROWS_PER_W + r
        base = b * S
        with jax.named_scope("gid"):
            pltpu.sync_copy(ids_h.at[pl.ds(base, SPLIT_A)], idx_a)
            pltpu.sync_copy(ids_h.at[pl.ds(base + SPLIT_A, SPLIT_B)], idx_b)
            pltpu.sync_copy(cidx_h.at[pl.ds(base, S)], cidx_v.at[pl.ds(0, S)])
        with jax.named_scope("gath"):
            ga = pltpu.async_copy(word_h.at[idx_a],
                                  acc_v.at[pl.ds(0, SPLIT_A)], gsem)
            gb = pltpu.async_copy(word_h.at[idx_b],
                                  acc_v.at[pl.ds(SPLIT_A, SPLIT_B)], gsem)
            ga.wait()
            gb.wait()

        with jax.named_scope("comp"):
            @plsc.parallel_loop(0, NQ, unroll=2)
            def group(q):
                t0 = q * LANES
                chunk = cidx_v[pl.ds(t0, LANES)]
                for i in range(LANES):
                    do_token(t0 + i, chunk[i])

            tail_chunk = cidx_v[pl.ds(NQ * LANES, LANES)]
            for i in range(TAIL):
                do_token(NQ * LANES + i, tail_chunk[i])

        with jax.named_scope("wb"):
            pltpu.sync_copy(acc_v, out_h.at[b])
        return carry
    lax.fori_loop(0, ROWS_PER_W, row, 0)


@jax.jit
def _run(ids, cidx, pos, typ, ten, word):
    mesh = plsc.VectorSubcoreMesh(core_axis_name="c", subcore_axis_name="s")
    return pl.kernel(
        _body,
        out_type=jax.ShapeDtypeStruct((B, S, H), jnp.float32),
        mesh=mesh,
        scratch_types=[
            pltpu.VMEM((S, H), jnp.float32),
            pltpu.VMEM((S, H), jnp.float32),
            pltpu.VMEM((2 * H,), jnp.float32),
            pltpu.VMEM((TEN_PAD, H), jnp.float32),
            pltpu.VMEM((S, H), jnp.float32),
            pltpu.VMEM((SPLIT_A,), jnp.int32),
            pltpu.VMEM((SPLIT_B,), jnp.int32),
            pltpu.VMEM(((NQ + 1) * LANES,), jnp.int32),
            pltpu.SemaphoreType.DMA,
        ],
    )(ids, cidx, pos, typ, ten, word)


def kernel(input_ids, token_type_ids, tenant_ids, W_word, W_pos, W_type, W_tenant):
    ids = input_ids.astype(jnp.int32).reshape(B * S)
    cidx = (token_type_ids.astype(jnp.int32) * 100
            + tenant_ids.astype(jnp.int32)).reshape(B * S)
    pos = W_pos[:S]
    typ = W_type.reshape(2 * H)
    ten = jnp.pad(W_tenant, ((0, TEN_PAD - W_tenant.shape[0]), (0, 0)))
    return _run(ids, cidx, pos, typ, ten, W_word)


# speedup vs baseline: 1.4584x; 1.4584x over previous
import jax
import jax.numpy as jnp
from jax import lax
from jax.experimental import pallas as pl
from jax.experimental.pallas import tpu as pltpu
from jax.experimental.pallas import tpu_sc as plsc

B = 1024
S = 200
H = 128
SPLIT_A = 104
SPLIT_B = S - SPLIT_A
NC = 2
NS = 16
NW = NC * NS
ROWS_PER_W = B // NW
LANES = 16
KCH = H // LANES
NQ = S // LANES
TAIL = S - NQ * LANES
TEN_PAD = 104


def _body(ids_h, cidx_h, pos_h, typ_h, ten_h, word_h, out_h,
          pos_v, combo_v, typ_v, ten_v, acc_v, idx_a, idx_b, cidx_v, gsem):
    c = lax.axis_index("c")
    s = lax.axis_index("s")
    wid = s * NC + c

    pltpu.sync_copy(pos_h, pos_v)
    pltpu.sync_copy(typ_h, typ_v)
    pltpu.sync_copy(ten_h, ten_v)

    def build(t, carry):
        for half in range(2):
            for k in range(KCH):
                sl = pl.ds(k * LANES, LANES)
                combo_v[half * 100 + t, sl] = (
                    ten_v[t, sl] + typ_v[pl.ds(half * H + k * LANES, LANES)])
        return carry
    lax.fori_loop(0, 100, build, 0)

    def do_token(t, ct):
        for k in range(KCH):
            sl = pl.ds(k * LANES, LANES)
            acc_v[t, sl] = acc_v[t, sl] + pos_v[t, sl] + combo_v[ct, sl]

    def row(r, carry):
        b = wid * ROWS_PER_W + r
        base = b * S
        with jax.named_scope("gid"):
            pltpu.sync_copy(ids_h.at[pl.ds(base, SPLIT_A)], idx_a)
            pltpu.sync_copy(ids_h.at[pl.ds(base + SPLIT_A, SPLIT_B)], idx_b)
            pltpu.sync_copy(cidx_h.at[pl.ds(base, S)], cidx_v.at[pl.ds(0, S)])
        with jax.named_scope("gath"):
            ga = pltpu.async_copy(word_h.at[idx_a],
                                  acc_v.at[pl.ds(0, SPLIT_A)], gsem)
            gb = pltpu.async_copy(word_h.at[idx_b],
                                  acc_v.at[pl.ds(SPLIT_A, SPLIT_B)], gsem)
            ga.wait()
            gb.wait()

        with jax.named_scope("comp"):
            @plsc.parallel_loop(0, NQ)
            def group(q):
                t0 = q * LANES
                chunk = cidx_v[pl.ds(t0, LANES)]
                for i in range(LANES):
                    do_token(t0 + i, chunk[i])

            tail_chunk = cidx_v[pl.ds(NQ * LANES, LANES)]
            for i in range(TAIL):
                do_token(NQ * LANES + i, tail_chunk[i])

        with jax.named_scope("wb"):
            pltpu.sync_copy(acc_v, out_h.at[b])
        return carry
    lax.fori_loop(0, ROWS_PER_W, row, 0)


@jax.jit
def _run(ids, cidx, pos, typ, ten, word):
    mesh = plsc.VectorSubcoreMesh(core_axis_name="c", subcore_axis_name="s")
    return pl.kernel(
        _body,
        out_type=jax.ShapeDtypeStruct((B, S, H), jnp.float32),
        mesh=mesh,
        scratch_types=[
            pltpu.VMEM((S, H), jnp.float32),
            pltpu.VMEM((S, H), jnp.float32),
            pltpu.VMEM((2 * H,), jnp.float32),
            pltpu.VMEM((TEN_PAD, H), jnp.float32),
            pltpu.VMEM((S, H), jnp.float32),
            pltpu.VMEM((SPLIT_A,), jnp.int32),
            pltpu.VMEM((SPLIT_B,), jnp.int32),
            pltpu.VMEM(((NQ + 1) * LANES,), jnp.int32),
            pltpu.SemaphoreType.DMA,
        ],
    )(ids, cidx, pos, typ, ten, word)


def kernel(input_ids, token_type_ids, tenant_ids, W_word, W_pos, W_type, W_tenant):
    ids = input_ids.astype(jnp.int32).reshape(B * S)
    cidx = (token_type_ids.astype(jnp.int32) * 100
            + tenant_ids.astype(jnp.int32)).reshape(B * S)
    pos = W_pos[:S]
    typ = W_type.reshape(2 * H)
    ten = jnp.pad(W_tenant, ((0, TEN_PAD - W_tenant.shape[0]), (0, 0)))
    return _run(ids, cidx, pos, typ, ten, W_word)


# speedup vs baseline: 1.5523x; 1.0644x over previous
import jax
import jax.numpy as jnp
from jax import lax
from jax.experimental import pallas as pl
from jax.experimental.pallas import tpu as pltpu
from jax.experimental.pallas import tpu_sc as plsc

B = 1024
S = 200
H = 128
SPLIT_A = 104
SPLIT_B = S - SPLIT_A
NC = 2
NS = 16
NW = NC * NS
ROWS_PER_W = B // NW
LANES = 16
KCH = H // LANES
NQ = S // LANES
TAIL = S - NQ * LANES
TEN_PAD = 104


def _body(ids_h, cidx_h, pos_h, typ_h, ten_h, word_h, out_h,
          pos_v, combo_v, typ_v, ten_v, acc0, acc1,
          ia0, ib0, ia1, ib1, cidx0, cidx1, g0, g1):
    c = lax.axis_index("c")
    s = lax.axis_index("s")
    wid = s * NC + c

    pltpu.sync_copy(pos_h, pos_v)
    pltpu.sync_copy(typ_h, typ_v)
    pltpu.sync_copy(ten_h, ten_v)

    def build(t, carry):
        for half in range(2):
            for k in range(KCH):
                sl = pl.ds(k * LANES, LANES)
                combo_v[half * 100 + t, sl] = (
                    ten_v[t, sl] + typ_v[pl.ds(half * H + k * LANES, LANES)])
        return carry
    lax.fori_loop(0, 100, build, 0)

    def prep_issue(r, ia, ib, cv, acc, sem):
        base = (wid * ROWS_PER_W + r) * S
        pltpu.sync_copy(ids_h.at[pl.ds(base, SPLIT_A)], ia)
        pltpu.sync_copy(ids_h.at[pl.ds(base + SPLIT_A, SPLIT_B)], ib)
        pltpu.sync_copy(cidx_h.at[pl.ds(base, S)], cv.at[pl.ds(0, S)])
        ga = pltpu.async_copy(word_h.at[ia], acc.at[pl.ds(0, SPLIT_A)], sem)
        gb = pltpu.async_copy(word_h.at[ib],
                              acc.at[pl.ds(SPLIT_A, SPLIT_B)], sem)
        return ga, gb

    def comp_wb(r, acc, cv):
        def do_token(t, ct):
            for k in range(KCH):
                sl = pl.ds(k * LANES, LANES)
                acc[t, sl] = acc[t, sl] + pos_v[t, sl] + combo_v[ct, sl]

        @plsc.parallel_loop(0, NQ)
        def group(q):
            t0 = q * LANES
            chunk = cv[pl.ds(t0, LANES)]
            for i in range(LANES):
                do_token(t0 + i, chunk[i])

        tail_chunk = cv[pl.ds(NQ * LANES, LANES)]
        for i in range(TAIL):
            do_token(NQ * LANES + i, tail_chunk[i])

        pltpu.sync_copy(acc, out_h.at[wid * ROWS_PER_W + r])

    ga, gb = prep_issue(0, ia0, ib0, cidx0, acc0, g0)
    ga.wait()
    gb.wait()

    def pair(p, carry):
        e = 2 * p
        d1a, d1b = prep_issue(e + 1, ia1, ib1, cidx1, acc1, g1)
        comp_wb(e, acc0, cidx0)
        d1a.wait()
        d1b.wait()
        d0a, d0b = prep_issue(e + 2, ia0, ib0, cidx0, acc0, g0)
        comp_wb(e + 1, acc1, cidx1)
        d0a.wait()
        d0b.wait()
        return carry
    lax.fori_loop(0, ROWS_PER_W // 2, pair, 0)


@jax.jit
def _run(ids, cidx, pos, typ, ten, word):
    mesh = plsc.VectorSubcoreMesh(core_axis_name="c", subcore_axis_name="s")
    return pl.kernel(
        _body,
        out_type=jax.ShapeDtypeStruct((B, S, H), jnp.float32),
        mesh=mesh,
        scratch_types=[
            pltpu.VMEM((S, H), jnp.float32),
            pltpu.VMEM((S, H), jnp.float32),
            pltpu.VMEM((2 * H,), jnp.float32),
            pltpu.VMEM((TEN_PAD, H), jnp.float32),
            pltpu.VMEM((S, H), jnp.float32),
            pltpu.VMEM((S, H), jnp.float32),
            pltpu.VMEM((SPLIT_A,), jnp.int32),
            pltpu.VMEM((SPLIT_B,), jnp.int32),
            pltpu.VMEM((SPLIT_A,), jnp.int32),
            pltpu.VMEM((SPLIT_B,), jnp.int32),
            pltpu.VMEM(((NQ + 1) * LANES,), jnp.int32),
            pltpu.VMEM(((NQ + 1) * LANES,), jnp.int32),
            pltpu.SemaphoreType.DMA,
            pltpu.SemaphoreType.DMA,
        ],
    )(ids, cidx, pos, typ, ten, word)


def kernel(input_ids, token_type_ids, tenant_ids, W_word, W_pos, W_type, W_tenant):
    ids = input_ids.astype(jnp.int32).reshape(B * S)
    cidx = (token_type_ids.astype(jnp.int32) * 100
            + tenant_ids.astype(jnp.int32)).reshape(B * S)
    pad = jnp.zeros((S,), jnp.int32)
    ids = jnp.concatenate([ids, pad])
    cidx = jnp.concatenate([cidx, pad])
    pos = W_pos[:S]
    typ = W_type.reshape(2 * H)
    ten = jnp.pad(W_tenant, ((0, TEN_PAD - W_tenant.shape[0]), (0, 0)))
    return _run(ids, cidx, pos, typ, ten, W_word)


# speedup vs baseline: 1.5977x; 1.0292x over previous
import jax
import jax.numpy as jnp
from jax import lax
from jax.experimental import pallas as pl
from jax.experimental.pallas import tpu as pltpu
from jax.experimental.pallas import tpu_sc as plsc

B = 1024
S = 200
H = 128
SPLIT_A = 104
SPLIT_B = S - SPLIT_A
NC = 2
NS = 16
NW = NC * NS
ROWS_PER_W = B // NW
LANES = 16
KCH = H // LANES
NQ = S // LANES
TAIL = S - NQ * LANES
TEN_PAD = 104


def _body(ids_h, cidx_h, pos_h, typ_h, ten_h, word_h, out_h,
          pos_v, combo_v, typ_v, ten_v, acc0, acc1,
          ia0, ib0, ia1, ib1, cidx0, cidx1, g0, g1, w0):
    c = lax.axis_index("c")
    s = lax.axis_index("s")
    wid = s * NC + c

    pltpu.sync_copy(pos_h, pos_v)
    pltpu.sync_copy(typ_h, typ_v)
    pltpu.sync_copy(ten_h, ten_v)

    def build(t, carry):
        for half in range(2):
            for k in range(KCH):
                sl = pl.ds(k * LANES, LANES)
                combo_v[half * 100 + t, sl] = (
                    ten_v[t, sl] + typ_v[pl.ds(half * H + k * LANES, LANES)])
        return carry
    lax.fori_loop(0, 100, build, 0)

    def prep(r, ia, ib, cv):
        base = (wid * ROWS_PER_W + r) * S
        pltpu.sync_copy(ids_h.at[pl.ds(base, SPLIT_A)], ia)
        pltpu.sync_copy(ids_h.at[pl.ds(base + SPLIT_A, SPLIT_B)], ib)
        pltpu.sync_copy(cidx_h.at[pl.ds(base, S)], cv.at[pl.ds(0, S)])

    def issue(ia, ib, acc, sem):
        ga = pltpu.async_copy(word_h.at[ia], acc.at[pl.ds(0, SPLIT_A)], sem)
        gb = pltpu.async_copy(word_h.at[ib],
                              acc.at[pl.ds(SPLIT_A, SPLIT_B)], sem)
        return ga, gb

    def prep_issue(r, ia, ib, cv, acc, sem):
        prep(r, ia, ib, cv)
        return issue(ia, ib, acc, sem)

    def comp(r, acc, cv):
        def do_token(t, ct):
            for k in range(KCH):
                sl = pl.ds(k * LANES, LANES)
                acc[t, sl] = acc[t, sl] + pos_v[t, sl] + combo_v[ct, sl]

        @plsc.parallel_loop(0, NQ)
        def group(q):
            t0 = q * LANES
            chunk = cv[pl.ds(t0, LANES)]
            for i in range(LANES):
                do_token(t0 + i, chunk[i])

        tail_chunk = cv[pl.ds(NQ * LANES, LANES)]
        for i in range(TAIL):
            do_token(NQ * LANES + i, tail_chunk[i])

    def comp_wb(r, acc, cv):
        comp(r, acc, cv)
        pltpu.sync_copy(acc, out_h.at[wid * ROWS_PER_W + r])

    ga, gb = prep_issue(0, ia0, ib0, cidx0, acc0, g0)
    ga.wait()
    gb.wait()

    def pair(p, carry):
        e = 2 * p
        d1a, d1b = prep_issue(e + 1, ia1, ib1, cidx1, acc1, g1)
        comp(e, acc0, cidx0)
        wb0 = pltpu.async_copy(acc0, out_h.at[wid * ROWS_PER_W + e], w0)
        d1a.wait()
        d1b.wait()
        prep(e + 2, ia0, ib0, cidx0)
        wb0.wait()
        d0a, d0b = issue(ia0, ib0, acc0, g0)
        comp_wb(e + 1, acc1, cidx1)
        d0a.wait()
        d0b.wait()
        return carry
    lax.fori_loop(0, ROWS_PER_W // 2, pair, 0)


@jax.jit
def _run(ids, cidx, pos, typ, ten, word):
    mesh = plsc.VectorSubcoreMesh(core_axis_name="c", subcore_axis_name="s")
    return pl.kernel(
        _body,
        out_type=jax.ShapeDtypeStruct((B, S, H), jnp.float32),
        mesh=mesh,
        scratch_types=[
            pltpu.VMEM((S, H), jnp.float32),
            pltpu.VMEM((S, H), jnp.float32),
            pltpu.VMEM((2 * H,), jnp.float32),
            pltpu.VMEM((TEN_PAD, H), jnp.float32),
            pltpu.VMEM((S, H), jnp.float32),
            pltpu.VMEM((S, H), jnp.float32),
            pltpu.VMEM((SPLIT_A,), jnp.int32),
            pltpu.VMEM((SPLIT_B,), jnp.int32),
            pltpu.VMEM((SPLIT_A,), jnp.int32),
            pltpu.VMEM((SPLIT_B,), jnp.int32),
            pltpu.VMEM(((NQ + 1) * LANES,), jnp.int32),
            pltpu.VMEM(((NQ + 1) * LANES,), jnp.int32),
            pltpu.SemaphoreType.DMA,
            pltpu.SemaphoreType.DMA,
            pltpu.SemaphoreType.DMA,
        ],
    )(ids, cidx, pos, typ, ten, word)


def kernel(input_ids, token_type_ids, tenant_ids, W_word, W_pos, W_type, W_tenant):
    ids = input_ids.astype(jnp.int32).reshape(B * S)
    cidx = (token_type_ids.astype(jnp.int32) * 100
            + tenant_ids.astype(jnp.int32)).reshape(B * S)
    pad = jnp.zeros((S,), jnp.int32)
    ids = jnp.concatenate([ids, pad])
    cidx = jnp.concatenate([cidx, pad])
    pos = W_pos[:S]
    typ = W_type.reshape(2 * H)
    ten = jnp.pad(W_tenant, ((0, TEN_PAD - W_tenant.shape[0]), (0, 0)))
    return _run(ids, cidx, pos, typ, ten, W_word)
